# baseline (speedup 1.0000x reference)
import jax
import jax.numpy as jnp
from jax import lax
from jax.experimental import pallas as pl
from jax.experimental.pallas import tpu as pltpu
from jax.experimental.pallas import tpu_sc as plsc

N = 50000
D = 128
E = 250000

NC = 2
NS = 16
L = 16

NPASS = 9
RNG = 5632
NPAD = NPASS * RNG
ACC_ROWS = RNG + 8
RPT = RNG // NS
DCH = 32
BLK = 2048
NBLK = 8

EP = 250112
EPT = EP // NS
VREGS = EPT // L
CAP = EPT + 2 * L

HALF = 25000
DEG_P = 25088
DEG_T = DEG_P // NS

MB = 99
RB = 512


def _vmesh():
    return plsc.VectorSubcoreMesh(
        core_axis_name="c", subcore_axis_name="s", num_cores=NC, num_subcores=NS
    )


def _deg_body(sd, deg_out, idx_v, stage_v, acc0, acc1, acc2, acc3, ones_v):
    c = lax.axis_index("c")
    w = lax.axis_index("s")
    zeros = jnp.zeros((L,), jnp.float32)

    def zero_body(i, _):
        stage_v[pl.ds(i * L, L)] = zeros
        return 0
    lax.fori_loop(0, DEG_T // L, zero_body, 0)
    for acc in (acc0, acc1, acc2, acc3):
        pltpu.sync_copy(stage_v, acc.at[pl.ds(w * DEG_T, DEG_T)])
    ones_v[...] = jnp.ones((L,), jnp.float32)
    plsc.subcore_barrier()

    lo = c * HALF
    lane = lax.iota(jnp.int32, L)
    dump = HALF + (lane & 7)
    for a, acc in enumerate((acc0, acc1, acc2, acc3)):
        pltpu.sync_copy(sd.at[pl.ds(a * EP + w * EPT, EPT)], idx_v)

        def cnt_body(i, _, acc=acc):
            v = idx_v[pl.ds(i * L, L)] - lo
            ok = (v >= 0) & (v < HALF)
            iv = jnp.where(ok, v, dump)
            pltpu.sync_copy(ones_v, acc.at[iv], add=True)
            return 0
        lax.fori_loop(0, VREGS, cnt_body, 0)
    plsc.subcore_barrier()

    for a, acc in enumerate((acc0, acc1, acc2, acc3)):
        pltpu.sync_copy(acc.at[pl.ds(w * DEG_T, DEG_T)], stage_v)
        pltpu.sync_copy(
            stage_v,
            deg_out.at[pl.ds((a * NC + c) * DEG_P + w * DEG_T, DEG_T)])


def _sc_degrees(sd):
    f = pl.kernel(
        _deg_body,
        out_type=jax.ShapeDtypeStruct((4 * NC * DEG_P,), jnp.float32),
        mesh=_vmesh(),
        scratch_types=[
            pltpu.VMEM((EPT,), jnp.int32),
            pltpu.VMEM((DEG_T,), jnp.float32),
            pltpu.VMEM_SHARED((DEG_P,), jnp.float32),
            pltpu.VMEM_SHARED((DEG_P,), jnp.float32),
            pltpu.VMEM_SHARED((DEG_P,), jnp.float32),
            pltpu.VMEM_SHARED((DEG_P,), jnp.float32),
            pltpu.VMEM((L,), jnp.float32),
        ],
    )
    return f(sd)


GROUP = 128
ROWS2 = 128
TRASH = 126 * GROUP


def _agg_body(hh, sd, agg, sidx, didx, csrc2, cdst2, buf0, buf1,
              zbuf, dbuf, cntb, gsem0, gsem1, acc):
    c = lax.axis_index("c")
    w = lax.axis_index("s")
    zeros = jnp.zeros((L,), jnp.float32)
    lane = lax.iota(jnp.int32, L)
    hbase = c * NPAD
    sbase = 2 * c * EP + w * EPT

    def zbody(i, _):
        for jc in range(8):
            zbuf[i, pl.ds(jc * L, L)] = zeros
        return 0
    lax.fori_loop(0, DCH, zbody, 0)

    pad_vec = hbase + jnp.full((L,), N, jnp.int32)

    def ibody(r, _):
        for jc in range(8):
            csrc2[r, pl.ds(jc * L, L)] = pad_vec
        return 0
    lax.fori_loop(0, ROWS2, ibody, 0)

    def one_pass(p, _):
        base = p * RNG

        for k in range(11):
            pltpu.sync_copy(zbuf, acc.at[pl.ds(w * RPT + k * DCH, DCH)])
        plsc.subcore_barrier()

        def blk_body(blk, cnt):
            pltpu.sync_copy(sd.at[pl.ds(sbase + blk * BLK, BLK)], sidx)
            pltpu.sync_copy(sd.at[pl.ds(sbase + EP + blk * BLK, BLK)], didx)
            nv = jnp.minimum(BLK // L, VREGS - blk * (BLK // L))

            def compact_body(i, cnt):
                s = sidx[pl.ds(i * L, L)]
                dloc = didx[pl.ds(i * L, L)] - base
                ok = (dloc >= 0) & (dloc < RNG)
                pos = jnp.where(ok, cnt * L + lane, TRASH + lane)
                pr = lax.shift_right_logical(pos, 7)
                pc = pos & 127
                plsc.store_scatter(csrc2, [pr, pc], s + hbase)
                plsc.store_scatter(cdst2, [pr, pc], dloc)
                return cnt + jnp.where(ok, 1, 0)
            return lax.fori_loop(0, nv, compact_body, cnt)
        cnt = lax.fori_loop(0, NBLK, blk_body,
                            jnp.zeros((L,), jnp.int32))

        maxv = cnt
        for kk in (1, 2, 4, 8):
            maxv = jnp.maximum(
                maxv, maxv.at[lane ^ kk].get(mode='promise_in_bounds'))
        cntb[...] = maxv
        maxc = cntb[pl.ds(0, L)][0]
        g2n = jnp.maximum((maxc + 15) // 16, 1)
        geff = g2n * 2

        dumpd = RNG + (lane & 7)

        def san_body(j, jv):
            r = lax.shift_right_logical(j, 3)
            cs = (j & 7) * L
            v = cdst2[r, pl.ds(cs, L)]
            cdst2[r, pl.ds(cs, L)] = jnp.where(jv < cnt, v, dumpd)
            return jv + 1
        lax.fori_loop(0, geff * 8, san_body, jnp.zeros((L,), jnp.int32))

        pltpu.async_copy(hh.at[csrc2.at[0]], buf0, gsem0)
        pltpu.async_copy(hh.at[csrc2.at[1]], buf1, gsem1)

        def gs_body(g2, _):
            for b, (buf, gs) in enumerate(((buf0, gsem0), (buf1, gsem1))):
                g = g2 * 2 + b
                pltpu.make_async_copy(hh.at[pl.ds(0, GROUP)], buf, gs).wait()
                pltpu.sync_copy(buf, acc.at[cdst2.at[g]], add=True)
                pltpu.async_copy(hh.at[csrc2.at[g + 2]], buf, gs)
            return 0
        lax.fori_loop(0, g2n, gs_body, 0)
        pltpu.make_async_copy(hh.at[pl.ds(0, GROUP)], buf0, gsem0).wait()
        pltpu.make_async_copy(hh.at[pl.ds(0, GROUP)], buf1, gsem1).wait()
        plsc.subcore_barrier()

        for k in range(11):
            pltpu.sync_copy(acc.at[pl.ds(w * RPT + k * DCH, DCH)], dbuf)
            pltpu.sync_copy(
                dbuf, agg.at[pl.ds(hbase + base + w * RPT + k * DCH, DCH)])
        plsc.subcore_barrier()
        return 0

    lax.fori_loop(0, NPASS, one_pass, 0)


def _sc_aggregate(hh, sd):
    f = pl.kernel(
        _agg_body,
        out_type=jax.ShapeDtypeStruct((2 * NPAD, D), jnp.float32),
        mesh=_vmesh(),
        compiler_params=pltpu.CompilerParams(needs_layout_passes=False),
        scratch_types=[
            pltpu.VMEM((BLK,), jnp.int32),
            pltpu.VMEM((BLK,), jnp.int32),
            pltpu.VMEM((ROWS2, GROUP), jnp.int32),
            pltpu.VMEM((ROWS2, GROUP), jnp.int32),
            pltpu.VMEM((GROUP, D), jnp.float32),
            pltpu.VMEM((GROUP, D), jnp.float32),
            pltpu.VMEM((DCH, D), jnp.float32),
            pltpu.VMEM((DCH, D), jnp.float32),
            pltpu.VMEM((L,), jnp.int32),
            pltpu.SemaphoreType.DMA,
            pltpu.SemaphoreType.DMA,
            pltpu.VMEM_SHARED((ACC_ROWS, D), jnp.float32),
        ],
    )
    return f(hh, sd)


def _norms_body(deg_ref, out_ref):
    d = deg_ref[...]
    out_ref[...] = lax.rsqrt(jnp.where(d > 0.0, d, 1.0))


def _tc_norms(deg):
    return pl.pallas_call(
        _norms_body,
        out_shape=jax.ShapeDtypeStruct((4, NPAD), jnp.float32),
    )(deg)


def _mm_body(x_ref, ns_ref, w_ref, out_ref):
    scale = jnp.reshape(ns_ref[...], (RB, 1))
    xs = x_ref[...] * scale
    out_ref[...] = jnp.dot(
        xs, jnp.reshape(w_ref[...], (D, D)), preferred_element_type=jnp.float32)


def _tc_matmul2(x, ns_stack, w_stack):
    return pl.pallas_call(
        _mm_body,
        out_shape=jax.ShapeDtypeStruct((2 * NPAD, D), jnp.float32),
        grid=(2, 98),
        in_specs=[
            pl.BlockSpec((RB, D), lambda r, i: (i, 0)),
            pl.BlockSpec((1, 1, RB, 1), lambda r, i: (r, i, 0, 0)),
            pl.BlockSpec((1, D, D), lambda r, i: (r, 0, 0)),
        ],
        out_specs=pl.BlockSpec((RB, D), lambda r, i: (r * MB + i, 0)),
    )(x, ns_stack, w_stack)


def _comb_body(a0_ref, a1_ref, n0_ref, n1_ref, b_ref, out_ref):
    n0 = jnp.reshape(n0_ref[...], (RB, 1))
    n1 = jnp.reshape(n1_ref[...], (RB, 1))
    h = a0_ref[...] * n0 + a1_ref[...] * n1 + b_ref[...]
    out_ref[...] = jnp.maximum(h, 0.0)


def _tc_combine(agg, nd_stack, b, out_rows, nblocks):
    return pl.pallas_call(
        _comb_body,
        out_shape=jax.ShapeDtypeStruct((out_rows, D), jnp.float32),
        grid=(nblocks,),
        in_specs=[
            pl.BlockSpec((RB, D), lambda i: (i, 0)),
            pl.BlockSpec((RB, D), lambda i: (MB + i, 0)),
            pl.BlockSpec((1, 1, RB, 1), lambda i: (0, i, 0, 0)),
            pl.BlockSpec((1, 1, RB, 1), lambda i: (1, i, 0, 0)),
            pl.BlockSpec((1, D), lambda i: (0, 0)),
        ],
        out_specs=pl.BlockSpec((RB, D), lambda i: (i, 0)),
    )(agg, agg, nd_stack, nd_stack, b)


def kernel(x, edge_index_r0, edge_index_r1, W1_0, b1_0, W1_1, b1_1,
           W2_0, b2_0, W2_1, b2_1):
    pad = jnp.full((EP - E,), N, jnp.int32)
    tail = jnp.full((BLK,), N, jnp.int32)
    sd = jnp.concatenate([
        edge_index_r0[0].astype(jnp.int32), pad,
        edge_index_r0[1].astype(jnp.int32), pad,
        edge_index_r1[0].astype(jnp.int32), pad,
        edge_index_r1[1].astype(jnp.int32), pad, tail,
    ])

    deg = _sc_degrees(sd).reshape(4, NC, DEG_P)
    deg_full = jnp.concatenate([deg[:, 0, :HALF], deg[:, 1, :HALF]], axis=1)
    deg_full = jnp.pad(deg_full, ((0, 0), (0, NPAD - N)))
    norms = _tc_norms(deg_full)
    ns_stack = norms[0::2].reshape(2, MB, RB, 1)
    nd_stack = norms[1::2].reshape(2, MB, RB, 1)
    w1_stack = jnp.stack([W1_0, W1_1])
    w2_stack = jnp.stack([W2_0, W2_1])
    bias1 = (b1_0 + b1_1).reshape(1, D)
    bias2 = (b2_0 + b2_1).reshape(1, D)

    h1 = _tc_matmul2(x, ns_stack, w1_stack)
    a1 = _sc_aggregate(h1, sd)
    x1 = _tc_combine(a1, nd_stack, bias1, NPAD, MB)

    h2 = _tc_matmul2(x1, ns_stack, w2_stack)
    a2 = _sc_aggregate(h2, sd)
    return _tc_combine(a2, nd_stack, bias2, N, 98)
